# baseline (speedup 1.0000x reference)
import jax
import jax.numpy as jnp
from jax import lax
from jax.experimental import pallas as pl
from jax.experimental.pallas import tpu as pltpu
from jax.experimental.pallas import tpu_sc as plsc

NUM_CORES = 2
NUM_SUBCORES = 16
NUM_WORKERS = NUM_CORES * NUM_SUBCORES
NUM_CHUNKS = 4


def kernel(tokens, embed_table):
    num_tokens = tokens.shape[0]
    dim = embed_table.shape[1]
    b_per_w = num_tokens // NUM_WORKERS
    chunk = b_per_w // NUM_CHUNKS

    mesh = plsc.VectorSubcoreMesh(core_axis_name="c", subcore_axis_name="s")

    @jax.jit
    def run(tok, table):
        @pl.kernel(
            mesh=mesh,
            out_type=jax.ShapeDtypeStruct((num_tokens, dim), table.dtype),
            scratch_types=[
                pltpu.VMEM((b_per_w,), jnp.int32),
                pltpu.VMEM((chunk, dim), table.dtype),
                pltpu.VMEM((chunk, dim), table.dtype),
                pltpu.SemaphoreType.DMA,
                pltpu.SemaphoreType.DMA,
                pltpu.SemaphoreType.DMA,
                pltpu.SemaphoreType.DMA,
            ],
        )
        def sc_gather(idx_hbm, table_hbm, out_hbm, idx_v, buf0, buf1,
                      gsem0, gsem1, wsem0, wsem1):
            wid = lax.axis_index("s") * NUM_CORES + lax.axis_index("c")
            base = wid * b_per_w
            pltpu.sync_copy(idx_hbm.at[pl.ds(base, b_per_w)], idx_v)

            bufs = (buf0, buf1)
            gsems = (gsem0, gsem1)
            wsems = (wsem0, wsem1)

            def gather(i):
                return pltpu.async_copy(
                    table_hbm.at[idx_v.at[pl.ds(i * chunk, chunk)]],
                    bufs[i % 2], gsems[i % 2])

            def writeback(i):
                return pltpu.async_copy(
                    bufs[i % 2], out_hbm.at[pl.ds(base + i * chunk, chunk)],
                    wsems[i % 2])

            gathers = [gather(0)]
            writes = []
            for i in range(NUM_CHUNKS):
                gathers[i].wait()
                if i + 1 < NUM_CHUNKS:
                    if i >= 1:
                        writes[i - 1].wait()
                    gathers.append(gather(i + 1))
                writes.append(writeback(i))
            writes[-2].wait()
            writes[-1].wait()

        return sc_gather(tok, table)

    return run(tokens.astype(jnp.int32), embed_table)


# speedup vs baseline: 1.0654x; 1.0654x over previous
import jax
import jax.numpy as jnp
from jax import lax
from jax.experimental import pallas as pl
from jax.experimental.pallas import tpu as pltpu
from jax.experimental.pallas import tpu_sc as plsc

NUM_CORES = 2
NUM_SUBCORES = 16
NUM_WORKERS = NUM_CORES * NUM_SUBCORES
NUM_CHUNKS = 4


def kernel(tokens, embed_table):
    num_tokens = tokens.shape[0]
    dim = embed_table.shape[1]
    b_per_w = num_tokens // NUM_WORKERS
    chunk = b_per_w // NUM_CHUNKS

    mesh = plsc.VectorSubcoreMesh(core_axis_name="c", subcore_axis_name="s")

    @jax.jit
    def run(tok, table):
        @pl.kernel(
            mesh=mesh,
            out_type=jax.ShapeDtypeStruct((num_tokens, dim), table.dtype),
            scratch_types=[
                pltpu.VMEM((b_per_w,), jnp.int32),
                pltpu.VMEM((b_per_w, dim), table.dtype),
            ] + [pltpu.SemaphoreType.DMA] * (NUM_CHUNKS + 1),
        )
        def sc_gather(idx_hbm, table_hbm, out_hbm, idx_v, rows_v, *sems):
            gsems, wsem = sems[:NUM_CHUNKS], sems[NUM_CHUNKS]
            wid = lax.axis_index("s") * NUM_CORES + lax.axis_index("c")
            base = wid * b_per_w
            pltpu.sync_copy(idx_hbm.at[pl.ds(base, b_per_w)], idx_v)

            gathers = [
                pltpu.async_copy(
                    table_hbm.at[idx_v.at[pl.ds(i * chunk, chunk)]],
                    rows_v.at[pl.ds(i * chunk, chunk)], gsems[i])
                for i in range(NUM_CHUNKS)
            ]
            writes = []
            for i in range(NUM_CHUNKS):
                gathers[i].wait()
                writes.append(pltpu.async_copy(
                    rows_v.at[pl.ds(i * chunk, chunk)],
                    out_hbm.at[pl.ds(base + i * chunk, chunk)], wsem))
            for w in writes:
                w.wait()

        return sc_gather(tok, table)

    return run(tokens.astype(jnp.int32), embed_table)
